# baseline (speedup 1.0000x reference)
import functools

import jax
import jax.numpy as jnp
from jax import lax
from jax.experimental import pallas as pl
from jax.experimental.pallas import tpu as pltpu
from jax.experimental.pallas import tpu_sc as plsc

SEQ = 16384
TOK = 20
D = 64

_info = plsc.get_sparse_core_info()
NC, NS = _info.num_cores, _info.num_subcores
NW = NC * NS
SPW = SEQ // NW
CHUNK = 128
NSUB = SPW // CHUNK
NCHUNK = NSUB * TOK
NBUF = 4

_mesh = plsc.VectorSubcoreMesh(core_axis_name="c", subcore_axis_name="s")

_scratch = (
    [pltpu.VMEM((SPW, TOK), jnp.int32)]
    + [pltpu.VMEM((CHUNK,), jnp.int32) for _ in range(NBUF)]
    + [pltpu.VMEM((CHUNK, D), jnp.float32) for _ in range(NBUF)]
    + [pltpu.VMEM((D, CHUNK), jnp.float32) for _ in range(NBUF)]
    + [pltpu.SemaphoreType.DMA for _ in range(2 * NBUF)]
)

_IOTA = None


@functools.partial(
    pl.kernel,
    mesh=_mesh,
    out_type=jax.ShapeDtypeStruct((TOK, 8, SEQ // CHUNK, 8, CHUNK), jnp.float32),
    scratch_types=_scratch,
    compiler_params=pltpu.CompilerParams(
        use_tc_tiling_on_sc=False, needs_layout_passes=False
    ),
)
def _gather(tids_hbm, table_hbm, out_hbm, ids_v, *rest):
    idx = rest[:NBUF]
    rows = rest[NBUF : 2 * NBUF]
    blocks = rest[2 * NBUF : 3 * NBUF]
    sem_g = rest[3 * NBUF : 4 * NBUF]
    sem_w = rest[4 * NBUF :]

    wid = lax.axis_index("s") * NC + lax.axis_index("c")
    ws = wid * SPW
    pltpu.sync_copy(tids_hbm.at[pl.ds(ws, SPW), :], ids_v)

    iota = lax.iota(jnp.int32, 16)

    def chunk_ts(j):
        csub = j // TOK
        t = j - csub * TOK
        return t, csub

    def extract_idx(j, b):
        t, csub = chunk_ts(j)
        for k in range(8):
            v = plsc.load_gather(
                ids_v, [csub * CHUNK + k * 16 + iota, jnp.full((16,), t, jnp.int32)]
            )
            idx[b][pl.ds(k * 16, 16)] = v

    def fire_gather(b):
        pltpu.async_copy(table_hbm.at[idx[b]], rows[b], sem_g[b])

    def wait_gather(b):
        pltpu.make_async_copy(table_hbm.at[idx[b]], rows[b], sem_g[b]).wait()

    def transpose(b):
        def body(d, carry):
            col = jnp.full((16,), 0, jnp.int32) + d
            for l0 in range(0, CHUNK, 16):
                v = plsc.load_gather(rows[b], [l0 + iota, col])
                blocks[b][d, pl.ds(l0, 16)] = v
            return carry

        lax.fori_loop(0, D, body, 0)

    def fire_out(j, b):
        t, csub = chunk_ts(j)
        c = wid * NSUB + csub
        for r in range(8):
            pltpu.async_copy(
                blocks[b].at[pl.ds(8 * r, 8), :], out_hbm.at[t, r, c], sem_w[b]
            )

    def wait_out(j, b):
        t, csub = chunk_ts(j)
        c = wid * NSUB + csub
        for r in range(8):
            pltpu.make_async_copy(
                blocks[b].at[pl.ds(8 * r, 8), :], out_hbm.at[t, r, c], sem_w[b]
            ).wait()

    for b in range(NBUF):
        extract_idx(b, b)
        fire_gather(b)

    for b in range(NBUF):
        wait_gather(b)
        transpose(b)
        fire_out(b, b)
        extract_idx(b + NBUF, b)
        fire_gather(b)

    def outer(o, carry):
        for b in range(NBUF):
            j = o * NBUF + b
            wait_gather(b)
            wait_out(j - NBUF, b)
            transpose(b)
            fire_out(j, b)
            extract_idx(j + NBUF, b)
            fire_gather(b)
        return carry

    lax.fori_loop(1, NCHUNK // NBUF - 1, outer, 0)

    for b in range(NBUF):
        j = NCHUNK - NBUF + b
        wait_gather(b)
        wait_out(j - NBUF, b)
        transpose(b)
        fire_out(j, b)
    for b in range(NBUF):
        wait_out(NCHUNK - NBUF + b, b)


def kernel(token_ids, weight):
    out5 = _gather(token_ids.astype(jnp.int32), weight)
    return out5.transpose(2, 4, 0, 1, 3).reshape(SEQ, TOK, D)


# speedup vs baseline: 1.2946x; 1.2946x over previous
import functools

import jax
import jax.numpy as jnp
from jax import lax
from jax.experimental import pallas as pl
from jax.experimental.pallas import tpu as pltpu
from jax.experimental.pallas import tpu_sc as plsc
from jax.experimental.layout import Format, Layout, with_layout_constraint

SEQ = 16384
TOK = 20
D = 64
B = SEQ * TOK

_info = plsc.get_sparse_core_info()
NC, NS = _info.num_cores, _info.num_subcores
NW = NC * NS
SPW = SEQ // NW
CHUNK = 128
NCHUNK = B // NW // CHUNK
NBUF = 8

_mesh = plsc.VectorSubcoreMesh(core_axis_name="c", subcore_axis_name="s")

_scratch = (
    [pltpu.VMEM((SPW, TOK), jnp.int32)]
    + [pltpu.VMEM((CHUNK,), jnp.int32) for _ in range(NBUF)]
    + [pltpu.VMEM((CHUNK, D), jnp.float32) for _ in range(NBUF)]
    + [pltpu.SemaphoreType.DMA for _ in range(2 * NBUF)]
)


@functools.partial(
    pl.kernel,
    mesh=_mesh,
    out_type=jax.ShapeDtypeStruct((B, D), jnp.float32),
    scratch_types=_scratch,
    compiler_params=pltpu.CompilerParams(
        use_tc_tiling_on_sc=False, needs_layout_passes=False
    ),
)
def _gather(tids_hbm, table_hbm, out_hbm, ids_v, *rest):
    idx = rest[:NBUF]
    rows = rest[NBUF : 2 * NBUF]
    sem_g = rest[2 * NBUF : 3 * NBUF]
    sem_w = rest[3 * NBUF :]

    wid = lax.axis_index("s") * NC + lax.axis_index("c")
    ws = wid * SPW
    pltpu.sync_copy(tids_hbm.at[pl.ds(ws, SPW), :], ids_v)

    iota = lax.iota(jnp.int32, 16)

    def chunk_ts(j):
        csub = j // TOK
        t = j - csub * TOK
        return t, csub

    def extract_idx(j, b):
        t, csub = chunk_ts(j)
        col = jnp.full((16,), t, jnp.int32)
        for k in range(8):
            v = plsc.load_gather(ids_v, [csub * CHUNK + k * 16 + iota, col])
            idx[b][pl.ds(k * 16, 16)] = v

    def fire_gather(b):
        pltpu.async_copy(table_hbm.at[idx[b]], rows[b], sem_g[b])

    def wait_gather(b):
        pltpu.make_async_copy(table_hbm.at[idx[b]], rows[b], sem_g[b]).wait()

    def out_slice(j):
        t, csub = chunk_ts(j)
        return out_hbm.at[pl.ds(t * SEQ + ws + csub * CHUNK, CHUNK)]

    def fire_out(j, b):
        pltpu.async_copy(rows[b], out_slice(j), sem_w[b])

    def wait_out(j, b):
        pltpu.make_async_copy(rows[b], out_slice(j), sem_w[b]).wait()

    for b in range(NBUF):
        extract_idx(b, b)
        fire_gather(b)

    def outer(o, carry):
        for b in range(NBUF):
            j = o * NBUF + b
            wait_gather(b)
            fire_out(j, b)
            wait_out(j, b)
            extract_idx(j + NBUF, b)
            fire_gather(b)
        return carry

    lax.fori_loop(0, (NCHUNK - NBUF) // NBUF, outer, 0)

    for b in range(NBUF):
        j = NCHUNK - NBUF + b
        wait_gather(b)
        fire_out(j, b)
        wait_out(j, b)


def kernel(token_ids, weight):
    out2d = _gather(token_ids.astype(jnp.int32), weight)
    out3dT = out2d.reshape(TOK, SEQ, D)
    out3dT = with_layout_constraint(
        out3dT, Layout(major_to_minor=(0, 2, 1), tiling=((8, 128),))
    )
    return out3dT.transpose(1, 0, 2)


# speedup vs baseline: 1.3524x; 1.0446x over previous
import functools

import jax
import jax.numpy as jnp
from jax import lax
from jax.experimental import pallas as pl
from jax.experimental.pallas import tpu as pltpu
from jax.experimental.pallas import tpu_sc as plsc

SEQ = 16384
TOK = 20
D = 64
DP = 128
B = SEQ * TOK

_info = plsc.get_sparse_core_info()
NC, NS = _info.num_cores, _info.num_subcores
NW = NC * NS
SPW = SEQ // NW
CHUNK = 128
NCHUNK = B // NW // CHUNK
NBUF = 4

_mesh = plsc.VectorSubcoreMesh(core_axis_name="c", subcore_axis_name="s")

_scratch = (
    [pltpu.VMEM((SPW, TOK), jnp.int32)]
    + [pltpu.VMEM((CHUNK,), jnp.int32) for _ in range(NBUF)]
    + [pltpu.VMEM((CHUNK, DP), jnp.float32) for _ in range(NBUF)]
    + [pltpu.SemaphoreType.DMA for _ in range(2 * NBUF)]
)


@functools.partial(
    pl.kernel,
    mesh=_mesh,
    out_type=jax.ShapeDtypeStruct((B, D), jnp.float32),
    scratch_types=_scratch,
    compiler_params=pltpu.CompilerParams(
        use_tc_tiling_on_sc=False, needs_layout_passes=False
    ),
)
def _gather(tids_hbm, table_hbm, out_hbm, ids_v, *rest):
    idx = rest[:NBUF]
    rows = rest[NBUF : 2 * NBUF]
    sem_g = rest[2 * NBUF : 3 * NBUF]
    sem_w = rest[3 * NBUF :]

    wid = lax.axis_index("s") * NC + lax.axis_index("c")
    ws = wid * SPW
    pltpu.sync_copy(tids_hbm.at[pl.ds(ws, SPW), :], ids_v)

    iota = lax.iota(jnp.int32, 16)

    def chunk_ts(j):
        csub = j // TOK
        t = j - csub * TOK
        return t, csub

    def extract_idx(j, b):
        t, csub = chunk_ts(j)
        col = jnp.full((16,), t, jnp.int32)
        for k in range(8):
            v = plsc.load_gather(ids_v, [csub * CHUNK + k * 16 + iota, col])
            idx[b][pl.ds(k * 16, 16)] = v

    def fire_gather(b):
        pltpu.async_copy(table_hbm.at[idx[b]], rows[b], sem_g[b])

    def wait_gather(b):
        pltpu.make_async_copy(table_hbm.at[idx[b]], rows[b], sem_g[b]).wait()

    def out_slice(j):
        t, csub = chunk_ts(j)
        return out_hbm.at[pl.ds(t * SEQ + ws + csub * CHUNK, CHUNK)]

    def fire_out(j, b):
        pltpu.async_copy(rows[b].at[:, pl.ds(0, D)], out_slice(j), sem_w[b])

    def wait_out(j, b):
        pltpu.make_async_copy(
            rows[b].at[:, pl.ds(0, D)], out_slice(j), sem_w[b]
        ).wait()

    for b in range(NBUF):
        extract_idx(b, b)
        fire_gather(b)

    def outer(o, carry):
        for b in range(NBUF):
            j = o * NBUF + b
            wait_gather(b)
            fire_out(j, b)
            wait_out(j, b)
            extract_idx(j + NBUF, b)
            fire_gather(b)
        return carry

    lax.fori_loop(0, (NCHUNK - NBUF) // NBUF, outer, 0)

    for b in range(NBUF):
        j = NCHUNK - NBUF + b
        wait_gather(b)
        fire_out(j, b)
        wait_out(j, b)


def kernel(token_ids, weight):
    wp = jnp.pad(weight, ((0, 0), (0, DP - D)))
    out2d = _gather(token_ids.astype(jnp.int32), wp)
    return out2d.reshape(TOK, SEQ, D).transpose(1, 0, 2)


# speedup vs baseline: 1.7310x; 1.2799x over previous
import functools

import jax
import jax.numpy as jnp
from jax import lax
from jax.experimental import pallas as pl
from jax.experimental.pallas import tpu as pltpu
from jax.experimental.pallas import tpu_sc as plsc

SEQ = 16384
TOK = 20
D = 64
DP = 128
B = SEQ * TOK

_info = plsc.get_sparse_core_info()
NC, NS = _info.num_cores, _info.num_subcores
NW = NC * NS
SPW = SEQ // NW
CHUNK = 128
NCHUNK = B // NW // CHUNK
NBUF = 4

_mesh = plsc.VectorSubcoreMesh(core_axis_name="c", subcore_axis_name="s")

_scratch = (
    [pltpu.VMEM((SPW, TOK), jnp.int32)]
    + [pltpu.VMEM((CHUNK,), jnp.int32) for _ in range(NBUF)]
    + [pltpu.VMEM((CHUNK, DP), jnp.float32) for _ in range(NBUF)]
    + [pltpu.VMEM((D, CHUNK), jnp.float32) for _ in range(NBUF)]
    + [pltpu.SemaphoreType.DMA for _ in range(2 * NBUF)]
)


@functools.partial(
    pl.kernel,
    mesh=_mesh,
    out_type=jax.ShapeDtypeStruct((TOK, 8, SEQ // CHUNK, 8, CHUNK), jnp.float32),
    scratch_types=_scratch,
    compiler_params=pltpu.CompilerParams(
        use_tc_tiling_on_sc=False, needs_layout_passes=False
    ),
)
def _gather(tids_hbm, table_hbm, out_hbm, ids_v, *rest):
    idx = rest[:NBUF]
    rows = rest[NBUF : 2 * NBUF]
    blocks = rest[2 * NBUF : 3 * NBUF]
    sem_g = rest[3 * NBUF : 4 * NBUF]
    sem_w = rest[4 * NBUF :]

    wid = lax.axis_index("s") * NC + lax.axis_index("c")
    ws = wid * SPW
    pltpu.sync_copy(tids_hbm.at[pl.ds(ws, SPW), :], ids_v)

    iota = lax.iota(jnp.int32, 16)

    def chunk_ts(j):
        csub = j // TOK
        t = j - csub * TOK
        return t, csub

    def extract_idx(j, b):
        t, csub = chunk_ts(j)
        col = jnp.full((16,), t, jnp.int32)
        for k in range(8):
            v = plsc.load_gather(ids_v, [csub * CHUNK + k * 16 + iota, col])
            idx[b][pl.ds(k * 16, 16)] = v

    def fire_gather(b):
        pltpu.async_copy(table_hbm.at[idx[b]], rows[b], sem_g[b])

    def wait_gather(b):
        pltpu.make_async_copy(table_hbm.at[idx[b]], rows[b], sem_g[b]).wait()

    def transpose(b):
        rb = rows[b]
        bb = blocks[b]

        @functools.partial(plsc.parallel_loop, 0, D, unroll=8)
        def _(d):
            col = jnp.full((16,), d, jnp.int32)
            for l0 in range(0, CHUNK, 16):
                bb[d, pl.ds(l0, 16)] = plsc.load_gather(rb, [l0 + iota, col])

    def fire_out(j, b):
        t, csub = chunk_ts(j)
        c = wid * (SPW // CHUNK) + csub
        for r in range(8):
            pltpu.async_copy(
                blocks[b].at[pl.ds(8 * r, 8), :], out_hbm.at[t, r, c], sem_w[b]
            )

    def wait_out(j, b):
        t, csub = chunk_ts(j)
        c = wid * (SPW // CHUNK) + csub
        for r in range(8):
            pltpu.make_async_copy(
                blocks[b].at[pl.ds(8 * r, 8), :], out_hbm.at[t, r, c], sem_w[b]
            ).wait()

    for b in range(NBUF):
        extract_idx(b, b)
        fire_gather(b)

    def outer(o, carry):
        for b in range(NBUF):
            j = o * NBUF + b
            wait_gather(b)
            transpose(b)
            fire_out(j, b)
            wait_out(j, b)
            extract_idx(j + NBUF, b)
            fire_gather(b)
        return carry

    lax.fori_loop(0, (NCHUNK - NBUF) // NBUF, outer, 0)

    for b in range(NBUF):
        j = NCHUNK - NBUF + b
        wait_gather(b)
        transpose(b)
        fire_out(j, b)
        wait_out(j, b)


def kernel(token_ids, weight):
    wp = jnp.pad(weight, ((0, 0), (0, DP - D)))
    out5 = _gather(token_ids.astype(jnp.int32), wp)
    return out5.transpose(2, 4, 0, 1, 3).reshape(SEQ, TOK, D)
